# baseline (speedup 1.0000x reference)
import functools

import jax
import jax.numpy as jnp
from jax.experimental import pallas as pl
from jax.experimental.pallas import tpu as pltpu

_NE = 8
_R = 8
_SCALING = 1.0
_PAD = 80

_SORT_NET = (
    (0, 1), (2, 3), (4, 5), (6, 7),
    (0, 2), (1, 3), (4, 6), (5, 7),
    (1, 2), (5, 6),
    (0, 4), (1, 5), (2, 6), (3, 7),
    (2, 4), (3, 5),
    (1, 2), (3, 4), (5, 6),
)


def _moe_body(x_ref, p_ref, q_ref, rep_ref, noise_ref, bias_ref, o_ref):
    ner = _NE * _R
    t = x_ref.shape[0]
    h = t // 2
    for k in range(2):
        g = jnp.dot(x_ref[k * h:(k + 1) * h, :].astype(jnp.bfloat16),
                    p_ref[...], preferred_element_type=jnp.float32)

        rt = g[:, ner:ner + 2 * _NE].T + bias_ref[:, :h]
        logits = rt[:_NE, :]
        nlogits = rt[_NE:, :]
        softplus = (jnp.maximum(nlogits, 0.0)
                    + jnp.log1p(jnp.exp(-jnp.abs(nlogits))))
        noisy = logits + noise_ref[:, k * h:(k + 1) * h] * softplus

        m = jnp.max(noisy, axis=0, keepdims=True)
        e = jnp.exp(noisy - m)
        p = e / jnp.sum(e, axis=0, keepdims=True)

        rows = [p[i:i + 1, :] for i in range(_NE)]
        for a, b in _SORT_NET:
            hi = jnp.maximum(rows[a], rows[b])
            lo = jnp.minimum(rows[a], rows[b])
            rows[a], rows[b] = hi, lo
        w = jnp.concatenate(rows, axis=0)

        scale = jax.lax.dot_general(
            w, rep_ref[...], (((0,), (0,)), ((), ())),
            preferred_element_type=jnp.float32)
        o_ref[k * h:(k + 1) * h, :] = jnp.dot(
            (g[:, :ner] * scale).astype(jnp.bfloat16), q_ref[...],
            preferred_element_type=jnp.float32)


@functools.partial(jax.jit, static_argnames=())
def kernel(x, Wg, bg, Wn, bn, A, B):
    n_tokens, n_embed = x.shape
    ner = _NE * _R
    tile = 2048

    a_all = A.reshape(ner, n_embed)
    proj = jnp.concatenate([a_all, Wg, Wn], axis=0)
    proj = jnp.pad(proj, ((0, _PAD - ner - 2 * _NE), (0, 0))).T
    proj = proj.astype(jnp.bfloat16)

    b_all = ((B.transpose(0, 2, 1).reshape(ner, n_embed))
             * _SCALING).astype(jnp.bfloat16)

    col = jnp.arange(ner)[None, :]
    row = jnp.arange(_NE)[:, None]
    rep = ((col // _R) == row).astype(jnp.float32)

    bias = jnp.broadcast_to(
        jnp.concatenate([bg, bn]).reshape(2 * _NE, 1), (2 * _NE, tile))

    noise_t = jax.random.normal(
        jax.random.key(42), (n_tokens, _NE), jnp.float32).T

    grid = (n_tokens // tile,)
    return pl.pallas_call(
        _moe_body,
        grid=grid,
        in_specs=[
            pl.BlockSpec((tile, n_embed), lambda i: (i, 0)),
            pl.BlockSpec((n_embed, _PAD), lambda i: (0, 0)),
            pl.BlockSpec((ner, n_embed), lambda i: (0, 0)),
            pl.BlockSpec((_NE, ner), lambda i: (0, 0)),
            pl.BlockSpec((_NE, tile), lambda i: (0, i)),
            pl.BlockSpec((2 * _NE, tile), lambda i: (0, 0)),
        ],
        out_specs=pl.BlockSpec((tile, n_embed), lambda i: (i, 0)),
        out_shape=jax.ShapeDtypeStruct((n_tokens, n_embed), jnp.float32),
    )(x, proj, b_all, rep, noise_t, bias)


# speedup vs baseline: 1.0252x; 1.0252x over previous
import functools

import jax
import jax.numpy as jnp
from jax.experimental import pallas as pl

_NE = 8
_R = 8
_SCALING = 1.0
_PAD = 80

_SORT_NET = (
    (0, 1), (2, 3), (4, 5), (6, 7),
    (0, 2), (1, 3), (4, 6), (5, 7),
    (1, 2), (5, 6),
    (0, 4), (1, 5), (2, 6), (3, 7),
    (2, 4), (3, 5),
    (1, 2), (3, 4), (5, 6),
)


def _moe_body(x_ref, p_ref, q_ref, rep_ref, noise_ref, bias_ref, o_ref):
    ner = _NE * _R
    t = x_ref.shape[0]
    h = t // 2
    for k in range(2):
        g = jnp.dot(x_ref[k * h:(k + 1) * h, :], p_ref[...],
                    preferred_element_type=jnp.float32)

        rt = g[:, ner:ner + 2 * _NE].T + bias_ref[:, :h]
        logits = rt[:_NE, :]
        nlogits = rt[_NE:, :]
        softplus = (jnp.maximum(nlogits, 0.0)
                    + jnp.log1p(jnp.exp(-jnp.abs(nlogits))))
        noisy = logits + noise_ref[:, k * h:(k + 1) * h] * softplus

        m = jnp.max(noisy, axis=0, keepdims=True)
        e = jnp.exp(noisy - m)
        p = e / jnp.sum(e, axis=0, keepdims=True)

        rows = [p[i:i + 1, :] for i in range(_NE)]
        for a, b in _SORT_NET:
            hi = jnp.maximum(rows[a], rows[b])
            lo = jnp.minimum(rows[a], rows[b])
            rows[a], rows[b] = hi, lo
        w = jnp.concatenate(rows, axis=0)

        scale = jax.lax.dot_general(
            w, rep_ref[...], (((0,), (0,)), ((), ())),
            preferred_element_type=jnp.float32)
        o_ref[k * h:(k + 1) * h, :] = jnp.dot(
            g[:, :ner] * scale, q_ref[...],
            preferred_element_type=jnp.float32)


@functools.partial(jax.jit, static_argnames=())
def kernel(x, Wg, bg, Wn, bn, A, B):
    n_tokens, n_embed = x.shape
    ner = _NE * _R
    tile = 2048

    a_all = A.reshape(ner, n_embed)
    proj = jnp.concatenate([a_all, Wg, Wn], axis=0)
    proj = jnp.pad(proj, ((0, _PAD - ner - 2 * _NE), (0, 0))).T

    b_all = (B.transpose(0, 2, 1).reshape(ner, n_embed)) * _SCALING

    col = jnp.arange(ner)[None, :]
    row = jnp.arange(_NE)[:, None]
    rep = ((col // _R) == row).astype(jnp.float32)

    bias = jnp.broadcast_to(
        jnp.concatenate([bg, bn]).reshape(2 * _NE, 1), (2 * _NE, tile))

    noise_t = jax.random.normal(
        jax.random.key(42), (n_tokens, _NE), jnp.float32).T

    grid = (n_tokens // tile,)
    return pl.pallas_call(
        _moe_body,
        grid=grid,
        in_specs=[
            pl.BlockSpec((tile, n_embed), lambda i: (i, 0)),
            pl.BlockSpec((n_embed, _PAD), lambda i: (0, 0)),
            pl.BlockSpec((ner, n_embed), lambda i: (0, 0)),
            pl.BlockSpec((_NE, ner), lambda i: (0, 0)),
            pl.BlockSpec((_NE, tile), lambda i: (0, i)),
            pl.BlockSpec((2 * _NE, tile), lambda i: (0, 0)),
        ],
        out_specs=pl.BlockSpec((tile, n_embed), lambda i: (i, 0)),
        out_shape=jax.ShapeDtypeStruct((n_tokens, n_embed), jnp.float32),
    )(x, proj, b_all, rep, noise_t, bias)
